# baseline (speedup 1.0000x reference)
import functools

import jax
import jax.numpy as jnp
from jax import lax
from jax.experimental import pallas as pl
from jax.experimental.pallas import tpu as pltpu
from jax.experimental.pallas import tpu_sc as plsc

P_REAL = 16800
P = 16896
HP = P // 2
G = 32
B = 16
CH = 1408
NCHUNK = HP // CH
NV = CH // 16
NVH = HP // 16
OVERLAP_T = 0.35
NEGPOS = 7
LN2 = 0.6931471805599453


def _logf(x):
    xb = plsc.bitcast(x, jnp.int32)
    e = ((xb >> 23) & 0xFF) - 127
    m = plsc.bitcast((xb & 0x007FFFFF) | (127 << 23), jnp.float32)
    big = m > jnp.float32(1.4142135)
    m = jnp.where(big, m * jnp.float32(0.5), m)
    e = jnp.where(big, e + 1, e)
    z = (m - jnp.float32(1.0)) / (m + jnp.float32(1.0))
    z2 = z * z
    p = jnp.float32(2.0 / 9.0)
    p = p * z2 + jnp.float32(2.0 / 7.0)
    p = p * z2 + jnp.float32(2.0 / 5.0)
    p = p * z2 + jnp.float32(2.0 / 3.0)
    p = p * z2 + jnp.float32(2.0)
    return e.astype(jnp.float32) * jnp.float32(LN2) + z * p


def _lane_rotate(x, s):
    idx = (lax.iota(jnp.int32, 16) + s) & 15
    return jnp.take(x, idx)


def _red16(x, op):
    for s in (1, 2, 4, 8):
        x = op(x, _lane_rotate(x, s))
    return x


def _smooth_l1(pred, tgt):
    d = jnp.abs(pred - tgt)
    return jnp.where(d < jnp.float32(1.0),
                     jnp.float32(0.5) * d * d,
                     d - jnp.float32(0.5))


_mesh = plsc.VectorSubcoreMesh(core_axis_name="c", subcore_axis_name="s")


@functools.partial(
    pl.kernel,
    out_type=jax.ShapeDtypeStruct((32, 128), jnp.float32),
    mesh=_mesh,
    compiler_params=pltpu.CompilerParams(needs_layout_passes=False),
    scratch_types=[
        pltpu.VMEM((4, CH), jnp.float32),
        pltpu.VMEM((2, CH), jnp.float32),
        pltpu.VMEM((4, CH), jnp.float32),
        pltpu.VMEM((10, CH), jnp.float32),
        pltpu.VMEM((15, 128), jnp.float32),
        pltpu.VMEM((HP,), jnp.float32),
        pltpu.VMEM((HP,), jnp.int32),
        pltpu.VMEM((HP,), jnp.float32),
        pltpu.VMEM((G, 16), jnp.float32),
        pltpu.VMEM((G, 16), jnp.int32),
        pltpu.VMEM((48,), jnp.float32),
        pltpu.VMEM((128,), jnp.float32),
        pltpu.VMEM((64,), jnp.float32),
        pltpu.VMEM((64,), jnp.float32),
        pltpu.VMEM((16,), jnp.float32),
        pltpu.VMEM((16,), jnp.float32),
        pltpu.VMEM_SHARED((16, 64), jnp.float32),
        pltpu.VMEM_SHARED((16, 16), jnp.float32),
    ],
)
def _sc_forward(conf_h, loc_h, lm_h, pri_h, tgt_h, out_h,
                pri_b, conf_b, loc_b, lm_b, tgt_b,
                bov, bidx, ce, gmax, gidx, tarea, outv,
                stga, prta, stgb, prtb,
                sha, shb):
    cid = lax.axis_index("c")
    sid = lax.axis_index("s")
    b = cid * 8 + (sid & 7)
    half = sid >> 3
    lo = half * HP

    pltpu.sync_copy(tgt_h.at[b], tgt_b)

    for h in range(2):
        sl = pl.ds(h * 16, 16)
        t0 = tgt_b[0, sl]
        t1 = tgt_b[1, sl]
        t2 = tgt_b[2, sl]
        t3 = tgt_b[3, sl]
        tarea[sl] = (t2 - t0) * (t3 - t1)

    def init_g(g, _):
        gmax[g, :] = jnp.full((16,), -1.0, jnp.float32)
        gidx[g, :] = jnp.zeros((16,), jnp.int32)
        return 0
    lax.fori_loop(0, G, init_g, 0)

    def pass_a(ci, _):
        p0 = ci * CH
        pltpu.sync_copy(pri_h.at[:, pl.ds(lo + p0, CH)], pri_b)

        def vloop(v, _):
            base = v * 16
            pcx = pri_b[0, pl.ds(base, 16)]
            pcy = pri_b[1, pl.ds(base, 16)]
            pw = pri_b[2, pl.ds(base, 16)]
            ph = pri_b[3, pl.ds(base, 16)]
            px0 = pcx - pw * jnp.float32(0.5)
            py0 = pcy - ph * jnp.float32(0.5)
            px1 = pcx + pw * jnp.float32(0.5)
            py1 = pcy + ph * jnp.float32(0.5)
            parea = (px1 - px0) * (py1 - py0)
            pidx = (lo + p0 + base) + lax.iota(jnp.int32, 16)

            def gloop(g, carry):
                bestv, bestg = carry
                gw16 = pl.ds(g, 16)
                tx0 = tgt_b[0, gw16][0]
                ty0 = tgt_b[1, gw16][0]
                tx1 = tgt_b[2, gw16][0]
                ty1 = tgt_b[3, gw16][0]
                ta = tarea[gw16][0]
                iw = jnp.maximum(
                    jnp.minimum(px1, tx1) - jnp.maximum(px0, tx0),
                    jnp.float32(0.0))
                ih = jnp.maximum(
                    jnp.minimum(py1, ty1) - jnp.maximum(py0, ty0),
                    jnp.float32(0.0))
                inter = iw * ih
                ov = inter / (ta + parea - inter)
                up = ov > bestv
                bestv = jnp.where(up, ov, bestv)
                bestg = jnp.where(up, g, bestg)
                gm = gmax[g, :]
                u2 = ov > gm
                gmax[g, :] = jnp.where(u2, ov, gm)
                gidx[g, :] = jnp.where(u2, pidx, gidx[g, :])
                return bestv, bestg

            bestv, bestg = lax.fori_loop(
                0, G, gloop,
                (jnp.full((16,), -1.0, jnp.float32),
                 jnp.zeros((16,), jnp.int32)))
            bov[pl.ds(p0 + base, 16)] = bestv
            bidx[pl.ds(p0 + base, 16)] = bestg
            return 0

        lax.fori_loop(0, NV, vloop, 0)
        return 0

    lax.fori_loop(0, NCHUNK, pass_a, 0)

    def bp_red(g, carry):
        mlo, mhi, ilo, ihi = carry
        gm = gmax[g, :]
        m = _red16(gm, jnp.maximum)
        cand = jnp.where(gm == m, gidx[g, :], jnp.int32(0x7FFFFFFF))
        bpg = _red16(cand, jnp.minimum)[0]
        l16 = lax.iota(jnp.int32, 16)
        mlo = jnp.where(l16 == g, m[0], mlo)
        mhi = jnp.where(l16 == (g - 16), m[0], mhi)
        ilo = jnp.where(l16 == g, bpg, ilo)
        ihi = jnp.where(l16 == (g - 16), bpg, ihi)
        return mlo, mhi, ilo, ihi

    mx_lo, mx_hi, bp_lo, bp_hi = lax.fori_loop(
        0, G, bp_red,
        (jnp.zeros((16,), jnp.float32), jnp.zeros((16,), jnp.float32),
         jnp.zeros((16,), jnp.int32), jnp.zeros((16,), jnp.int32)))

    stga[pl.ds(0, 16)] = mx_lo
    stga[pl.ds(16, 16)] = mx_hi
    stga[pl.ds(32, 16)] = bp_lo.astype(jnp.float32)
    stga[pl.ds(48, 16)] = bp_hi.astype(jnp.float32)
    plsc.subcore_barrier()
    pltpu.sync_copy(stga, sha.at[sid])
    plsc.subcore_barrier()
    plsc.subcore_barrier()
    pltpu.sync_copy(sha.at[sid ^ 8], prta)
    plsc.subcore_barrier()
    p_mx_lo = prta[pl.ds(0, 16)]
    p_mx_hi = prta[pl.ds(16, 16)]
    p_bp_lo = prta[pl.ds(32, 16)].astype(jnp.int32)
    p_bp_hi = prta[pl.ds(48, 16)].astype(jnp.int32)

    is_low = half == 0
    lmx_lo = jnp.where(is_low, mx_lo, p_mx_lo)
    hmx_lo = jnp.where(is_low, p_mx_lo, mx_lo)
    lbp_lo = jnp.where(is_low, bp_lo, p_bp_lo)
    hbp_lo = jnp.where(is_low, p_bp_lo, bp_lo)
    bp_lo = jnp.where(lmx_lo >= hmx_lo, lbp_lo, hbp_lo)
    lmx_hi = jnp.where(is_low, mx_hi, p_mx_hi)
    hmx_hi = jnp.where(is_low, p_mx_hi, mx_hi)
    lbp_hi = jnp.where(is_low, bp_hi, p_bp_hi)
    hbp_hi = jnp.where(is_low, p_bp_hi, bp_hi)
    bp_hi = jnp.where(lmx_hi >= hmx_hi, lbp_hi, hbp_hi)

    l16 = lax.iota(jnp.int32, 16)
    in_lo = (bp_lo >= lo) & (bp_lo < lo + HP)
    in_hi = (bp_hi >= lo) & (bp_hi < lo + HP)
    loc_lo = jnp.clip(bp_lo - lo, 0, HP - 1)
    loc_hi = jnp.clip(bp_hi - lo, 0, HP - 1)
    two = jnp.full((16,), 2.0, jnp.float32)
    plsc.store_scatter(bov, [loc_lo], two, mask=in_lo)
    plsc.store_scatter(bov, [loc_hi], two, mask=in_hi)
    for j in range(16):
        plsc.store_scatter(bidx, [loc_lo], l16, mask=in_lo & (l16 == j))
    for j in range(16):
        plsc.store_scatter(bidx, [loc_hi], l16 + 16, mask=in_hi & (l16 == j))

    def pass_b(ci, carry):
        acc_box, acc_lm, acc_ce, acc_np = carry
        p0 = ci * CH
        pltpu.sync_copy(pri_h.at[:, pl.ds(lo + p0, CH)], pri_b)
        pltpu.sync_copy(conf_h.at[b, :, pl.ds(lo + p0, CH)], conf_b)
        pltpu.sync_copy(loc_h.at[b, :, pl.ds(lo + p0, CH)], loc_b)
        pltpu.sync_copy(lm_h.at[b, :, pl.ds(lo + p0, CH)], lm_b)

        def vloop(v, carry2):
            ab, al, ac, an = carry2
            base = v * 16
            sl = pl.ds(base, 16)
            gsl = pl.ds(p0 + base, 16)
            pcx = pri_b[0, sl]
            pcy = pri_b[1, sl]
            pw = pri_b[2, sl]
            ph = pri_b[3, sl]
            rw = jnp.float32(1.0) / pw
            rh = jnp.float32(1.0) / ph
            bid = bidx[gsl]
            pos = bov[gsl] >= jnp.float32(OVERLAP_T)
            posf = jnp.where(pos, jnp.float32(1.0), jnp.float32(0.0))

            def col(c):
                return plsc.load_gather(
                    tgt_b, [jnp.full((16,), c, jnp.int32), bid])

            m0 = col(0)
            m1 = col(1)
            m2 = col(2)
            m3 = col(3)
            gx = ((m0 + m2) * jnp.float32(0.5) - pcx) * rw * jnp.float32(10.0)
            gy = ((m1 + m3) * jnp.float32(0.5) - pcy) * rh * jnp.float32(10.0)
            gw = _logf((m2 - m0) * rw) * jnp.float32(5.0)
            gh = _logf((m3 - m1) * rh) * jnp.float32(5.0)
            lbox = (_smooth_l1(loc_b[0, sl], gx) +
                    _smooth_l1(loc_b[1, sl], gy) +
                    _smooth_l1(loc_b[2, sl], gw) +
                    _smooth_l1(loc_b[3, sl], gh))
            ab = ab + posf * lbox

            llm = jnp.zeros((16,), jnp.float32)
            for i in range(5):
                lx = col(4 + 2 * i)
                ly = col(5 + 2 * i)
                ex = (lx - pcx) * rw * jnp.float32(10.0)
                ey = (ly - pcy) * rh * jnp.float32(10.0)
                llm = (llm + _smooth_l1(lm_b[2 * i, sl], ex) +
                       _smooth_l1(lm_b[2 * i + 1, sl], ey))
            al = al + posf * llm

            x0 = conf_b[0, sl]
            x1 = conf_b[1, sl]
            z = jnp.where(pos, x0 - x1, x1 - x0)
            cev = (jnp.maximum(z, jnp.float32(0.0)) +
                   _logf(jnp.float32(1.0) + jnp.exp(-jnp.abs(z))))
            ac = ac + posf * cev
            an = an + posf
            valid = ((lo + p0 + base) + lax.iota(jnp.int32, 16)) < P_REAL
            ce[gsl] = jnp.where(pos | jnp.logical_not(valid),
                                jnp.float32(0.0), cev)
            return ab, al, ac, an

        return lax.fori_loop(0, NV, vloop, carry)

    zero = jnp.zeros((16,), jnp.float32)
    acc_box, acc_lm, acc_ce, acc_np = lax.fori_loop(
        0, NCHUNK, pass_b, (zero, zero, zero, zero))

    npos_f = _red16(acc_np, jnp.add)[0]
    loss_box = _red16(acc_box, jnp.add)[0]
    loss_lm = _red16(acc_lm, jnp.add)[0]
    ce_pos = _red16(acc_ce, jnp.add)[0]

    stgb[...] = jnp.full((16,), 1.0, jnp.float32) * npos_f
    plsc.subcore_barrier()
    pltpu.sync_copy(stgb, shb.at[sid])
    plsc.subcore_barrier()
    plsc.subcore_barrier()
    pltpu.sync_copy(shb.at[sid ^ 8], prtb)
    plsc.subcore_barrier()
    npos_tot = npos_f + prtb[pl.ds(0, 16)][0]

    k = jnp.minimum(npos_tot.astype(jnp.int32) * NEGPOS, P_REAL - 1)

    def bis(i, carry):
        lo_b, hi_b = carry
        mid = lo_b + ((hi_b - lo_b + 1) >> 1)
        midv = jnp.full((16,), 1, jnp.int32) * mid

        def cnt_loop(v, a):
            bits = plsc.bitcast(ce[pl.ds(v * 16, 16)], jnp.int32)
            return a + plsc.all_reduce_population_count(bits >= midv)

        cnt = lax.fori_loop(0, NVH, cnt_loop,
                            jnp.zeros((16,), jnp.int32))
        stgb[...] = jnp.full((16,), 1.0, jnp.float32) * cnt[0].astype(jnp.float32)
        plsc.subcore_barrier()
        pltpu.sync_copy(stgb, shb.at[sid])
        plsc.subcore_barrier()
        plsc.subcore_barrier()
        pltpu.sync_copy(shb.at[sid ^ 8], prtb)
        plsc.subcore_barrier()
        cnt_tot = cnt[0] + prtb[pl.ds(0, 16)][0].astype(jnp.int32)
        ge = cnt_tot >= k
        lo_b = jnp.where(ge, mid, lo_b)
        hi_b = jnp.where(ge, hi_b, mid - 1)
        return lo_b, hi_b

    lo_b, _ = lax.fori_loop(0, 31, bis,
                            (jnp.int32(0), jnp.int32(0x7F800000)))
    vkv = plsc.bitcast(jnp.full((16,), 1, jnp.int32) * lo_b, jnp.float32)

    def fin_loop(v, carry):
        sgt, cgt = carry
        x = ce[pl.ds(v * 16, 16)]
        gt = x > vkv
        sgt = sgt + jnp.where(gt, x, jnp.float32(0.0))
        cgt = cgt + plsc.all_reduce_population_count(gt)
        return sgt, cgt

    sgt, cgt = lax.fori_loop(
        0, NVH, fin_loop,
        (jnp.zeros((16,), jnp.float32), jnp.zeros((16,), jnp.int32)))
    sgt_sum = _red16(sgt, jnp.add)[0]

    li = lax.iota(jnp.int32, 16)
    fz = jnp.float32(0.0)
    o = (jnp.where(li == 0, npos_f, fz) +
         jnp.where(li == 1, loss_box, fz) +
         jnp.where(li == 2, ce_pos, fz) +
         jnp.where(li == 3, loss_lm, fz) +
         jnp.where(li == 4, sgt_sum, fz) +
         jnp.where(li == 5, cgt[0].astype(jnp.float32), fz) +
         jnp.where(li == 6, vkv, fz) +
         jnp.where(li == 7, k.astype(jnp.float32), fz))
    outv[pl.ds(0, 16)] = o
    pltpu.sync_copy(outv, out_h.at[cid * 16 + sid])


def kernel(conf_data, loc_data, landm_data, priors, targets):
    pad = P - P_REAL
    conf_t = jnp.pad(jnp.transpose(conf_data, (0, 2, 1)),
                     ((0, 0), (0, 0), (0, pad)))
    loc_t = jnp.pad(jnp.transpose(loc_data, (0, 2, 1)),
                    ((0, 0), (0, 0), (0, pad)))
    lm_t = jnp.pad(jnp.transpose(landm_data, (0, 2, 1)),
                   ((0, 0), (0, 0), (0, pad)))
    pri_pad = jnp.broadcast_to(
        jnp.array([[100.0], [100.0], [1.0], [1.0]], jnp.float32), (4, pad))
    pri_t = jnp.concatenate([jnp.transpose(priors, (1, 0)), pri_pad], axis=1)
    tgt_t = jnp.pad(jnp.transpose(targets, (0, 2, 1)),
                    ((0, 0), (0, 0), (0, 128 - G)))
    part = _sc_forward(conf_t, loc_t, lm_t, pri_t, tgt_t)
    v = part.reshape(2, 2, 8, 128)
    h0 = v[:, 0]
    h1 = v[:, 1]
    npos = jnp.maximum(jnp.sum(h0[..., 0] + h1[..., 0]), 1.0)
    loss_box = jnp.sum(h0[..., 1] + h1[..., 1])
    ce_pos = jnp.sum(h0[..., 2] + h1[..., 2])
    loss_lm = jnp.sum(h0[..., 3] + h1[..., 3])
    topk = jnp.sum(h0[..., 4] + h1[..., 4] +
                   (h0[..., 7] - h0[..., 5] - h1[..., 5]) * h0[..., 6])
    return (loss_box / npos, (ce_pos + topk) / npos, loss_lm / npos)


# speedup vs baseline: 1.0152x; 1.0152x over previous
import functools

import jax
import jax.numpy as jnp
from jax import lax
from jax.experimental import pallas as pl
from jax.experimental.pallas import tpu as pltpu
from jax.experimental.pallas import tpu_sc as plsc

P_REAL = 16800
P = 16896
HP = P // 2
G = 32
B = 16
CH = 1408
NCHUNK = HP // CH
NV = CH // 16
NVH = HP // 16
OVERLAP_T = 0.35
NEGPOS = 7
LN2 = 0.6931471805599453


def _logf(x):
    xb = plsc.bitcast(x, jnp.int32)
    e = ((xb >> 23) & 0xFF) - 127
    m = plsc.bitcast((xb & 0x007FFFFF) | (127 << 23), jnp.float32)
    big = m > jnp.float32(1.4142135)
    m = jnp.where(big, m * jnp.float32(0.5), m)
    e = jnp.where(big, e + 1, e)
    z = (m - jnp.float32(1.0)) / (m + jnp.float32(1.0))
    z2 = z * z
    p = jnp.float32(2.0 / 9.0)
    p = p * z2 + jnp.float32(2.0 / 7.0)
    p = p * z2 + jnp.float32(2.0 / 5.0)
    p = p * z2 + jnp.float32(2.0 / 3.0)
    p = p * z2 + jnp.float32(2.0)
    return e.astype(jnp.float32) * jnp.float32(LN2) + z * p


def _lane_rotate(x, s):
    idx = (lax.iota(jnp.int32, 16) + s) & 15
    return jnp.take(x, idx)


def _red16(x, op):
    for s in (1, 2, 4, 8):
        x = op(x, _lane_rotate(x, s))
    return x


def _smooth_l1(pred, tgt):
    d = jnp.abs(pred - tgt)
    return jnp.where(d < jnp.float32(1.0),
                     jnp.float32(0.5) * d * d,
                     d - jnp.float32(0.5))


_mesh = plsc.VectorSubcoreMesh(core_axis_name="c", subcore_axis_name="s")


@functools.partial(
    pl.kernel,
    out_type=jax.ShapeDtypeStruct((32, 128), jnp.float32),
    mesh=_mesh,
    compiler_params=pltpu.CompilerParams(needs_layout_passes=False),
    scratch_types=[
        pltpu.VMEM((4, HP), jnp.float32),
        pltpu.VMEM((2, CH), jnp.float32),
        pltpu.VMEM((4, CH), jnp.float32),
        pltpu.VMEM((10, CH), jnp.float32),
        pltpu.VMEM((15, 128), jnp.float32),
        pltpu.VMEM((HP,), jnp.float32),
        pltpu.VMEM((HP,), jnp.int32),
        pltpu.VMEM((HP,), jnp.float32),
        pltpu.VMEM((G, 16), jnp.float32),
        pltpu.VMEM((G, 16), jnp.int32),
        pltpu.VMEM((48,), jnp.float32),
        pltpu.VMEM((128,), jnp.float32),
        pltpu.VMEM((64,), jnp.float32),
        pltpu.VMEM((64,), jnp.float32),
        pltpu.VMEM((16,), jnp.float32),
        pltpu.VMEM((16,), jnp.float32),
        pltpu.VMEM_SHARED((16, 64), jnp.float32),
        pltpu.VMEM_SHARED((16, 16), jnp.float32),
    ],
)
def _sc_forward(conf_h, loc_h, lm_h, pri_h, tgt_h, out_h,
                pri_b, conf_b, loc_b, lm_b, tgt_b,
                bov, bidx, ce, gmax, gidx, tarea, outv,
                stga, prta, stgb, prtb,
                sha, shb):
    cid = lax.axis_index("c")
    sid = lax.axis_index("s")
    b = cid * 8 + (sid & 7)
    half = sid >> 3
    lo = half * HP

    pltpu.sync_copy(tgt_h.at[b], tgt_b)
    pltpu.sync_copy(pri_h.at[:, pl.ds(lo, HP)], pri_b)

    for h in range(2):
        sl = pl.ds(h * 16, 16)
        t0 = tgt_b[0, sl]
        t1 = tgt_b[1, sl]
        t2 = tgt_b[2, sl]
        t3 = tgt_b[3, sl]
        tarea[sl] = (t2 - t0) * (t3 - t1)

    def init_g(g, _):
        gmax[g, :] = jnp.full((16,), -1.0, jnp.float32)
        gidx[g, :] = jnp.zeros((16,), jnp.int32)
        return 0
    lax.fori_loop(0, G, init_g, 0)

    def vloop_a(v, _):
        base = v * 16
        pcx = pri_b[0, pl.ds(base, 16)]
        pcy = pri_b[1, pl.ds(base, 16)]
        pw = pri_b[2, pl.ds(base, 16)]
        ph = pri_b[3, pl.ds(base, 16)]
        px0 = pcx - pw * jnp.float32(0.5)
        py0 = pcy - ph * jnp.float32(0.5)
        px1 = pcx + pw * jnp.float32(0.5)
        py1 = pcy + ph * jnp.float32(0.5)
        parea = (px1 - px0) * (py1 - py0)
        pidx = (lo + base) + lax.iota(jnp.int32, 16)

        def gloop(g, carry):
            bestv, bestg = carry
            gw16 = pl.ds(g, 16)
            tx0 = tgt_b[0, gw16][0]
            ty0 = tgt_b[1, gw16][0]
            tx1 = tgt_b[2, gw16][0]
            ty1 = tgt_b[3, gw16][0]
            ta = tarea[gw16][0]
            iw = jnp.maximum(
                jnp.minimum(px1, tx1) - jnp.maximum(px0, tx0),
                jnp.float32(0.0))
            ih = jnp.maximum(
                jnp.minimum(py1, ty1) - jnp.maximum(py0, ty0),
                jnp.float32(0.0))
            inter = iw * ih
            ov = inter / (ta + parea - inter)
            up = ov > bestv
            bestv = jnp.where(up, ov, bestv)
            bestg = jnp.where(up, g, bestg)
            gm = gmax[g, :]
            u2 = ov > gm
            gmax[g, :] = jnp.where(u2, ov, gm)
            gidx[g, :] = jnp.where(u2, pidx, gidx[g, :])
            return bestv, bestg

        bestv, bestg = lax.fori_loop(
            0, G, gloop,
            (jnp.full((16,), -1.0, jnp.float32),
             jnp.zeros((16,), jnp.int32)))
        bov[pl.ds(base, 16)] = bestv
        bidx[pl.ds(base, 16)] = bestg
        return 0

    lax.fori_loop(0, NVH, vloop_a, 0)

    def bp_red(g, carry):
        mlo, mhi, ilo, ihi = carry
        gm = gmax[g, :]
        m = _red16(gm, jnp.maximum)
        cand = jnp.where(gm == m, gidx[g, :], jnp.int32(0x7FFFFFFF))
        bpg = _red16(cand, jnp.minimum)[0]
        l16 = lax.iota(jnp.int32, 16)
        mlo = jnp.where(l16 == g, m[0], mlo)
        mhi = jnp.where(l16 == (g - 16), m[0], mhi)
        ilo = jnp.where(l16 == g, bpg, ilo)
        ihi = jnp.where(l16 == (g - 16), bpg, ihi)
        return mlo, mhi, ilo, ihi

    mx_lo, mx_hi, bp_lo, bp_hi = lax.fori_loop(
        0, G, bp_red,
        (jnp.zeros((16,), jnp.float32), jnp.zeros((16,), jnp.float32),
         jnp.zeros((16,), jnp.int32), jnp.zeros((16,), jnp.int32)))

    stga[pl.ds(0, 16)] = mx_lo
    stga[pl.ds(16, 16)] = mx_hi
    stga[pl.ds(32, 16)] = bp_lo.astype(jnp.float32)
    stga[pl.ds(48, 16)] = bp_hi.astype(jnp.float32)
    plsc.subcore_barrier()
    pltpu.sync_copy(stga, sha.at[sid])
    plsc.subcore_barrier()
    plsc.subcore_barrier()
    pltpu.sync_copy(sha.at[sid ^ 8], prta)
    plsc.subcore_barrier()
    p_mx_lo = prta[pl.ds(0, 16)]
    p_mx_hi = prta[pl.ds(16, 16)]
    p_bp_lo = prta[pl.ds(32, 16)].astype(jnp.int32)
    p_bp_hi = prta[pl.ds(48, 16)].astype(jnp.int32)

    is_low = half == 0
    lmx_lo = jnp.where(is_low, mx_lo, p_mx_lo)
    hmx_lo = jnp.where(is_low, p_mx_lo, mx_lo)
    lbp_lo = jnp.where(is_low, bp_lo, p_bp_lo)
    hbp_lo = jnp.where(is_low, p_bp_lo, bp_lo)
    bp_lo = jnp.where(lmx_lo >= hmx_lo, lbp_lo, hbp_lo)
    lmx_hi = jnp.where(is_low, mx_hi, p_mx_hi)
    hmx_hi = jnp.where(is_low, p_mx_hi, mx_hi)
    lbp_hi = jnp.where(is_low, bp_hi, p_bp_hi)
    hbp_hi = jnp.where(is_low, p_bp_hi, bp_hi)
    bp_hi = jnp.where(lmx_hi >= hmx_hi, lbp_hi, hbp_hi)

    l16 = lax.iota(jnp.int32, 16)
    in_lo = (bp_lo >= lo) & (bp_lo < lo + HP)
    in_hi = (bp_hi >= lo) & (bp_hi < lo + HP)
    loc_lo = jnp.clip(bp_lo - lo, 0, HP - 1)
    loc_hi = jnp.clip(bp_hi - lo, 0, HP - 1)
    two = jnp.full((16,), 2.0, jnp.float32)
    plsc.store_scatter(bov, [loc_lo], two, mask=in_lo)
    plsc.store_scatter(bov, [loc_hi], two, mask=in_hi)
    for j in range(16):
        plsc.store_scatter(bidx, [loc_lo], l16, mask=in_lo & (l16 == j))
    for j in range(16):
        plsc.store_scatter(bidx, [loc_hi], l16 + 16, mask=in_hi & (l16 == j))

    def pass_b(ci, carry):
        acc_box, acc_lm, acc_ce, acc_np = carry
        p0 = ci * CH
        pltpu.sync_copy(conf_h.at[b, :, pl.ds(lo + p0, CH)], conf_b)
        pltpu.sync_copy(loc_h.at[b, :, pl.ds(lo + p0, CH)], loc_b)
        pltpu.sync_copy(lm_h.at[b, :, pl.ds(lo + p0, CH)], lm_b)

        def vloop(v, carry2):
            ab, al, ac, an = carry2
            base = v * 16
            sl = pl.ds(base, 16)
            gsl = pl.ds(p0 + base, 16)
            pcx = pri_b[0, gsl]
            pcy = pri_b[1, gsl]
            pw = pri_b[2, gsl]
            ph = pri_b[3, gsl]
            rw = jnp.float32(1.0) / pw
            rh = jnp.float32(1.0) / ph
            bid = bidx[gsl]
            pos = bov[gsl] >= jnp.float32(OVERLAP_T)
            posf = jnp.where(pos, jnp.float32(1.0), jnp.float32(0.0))

            def col(c):
                return plsc.load_gather(
                    tgt_b, [jnp.full((16,), c, jnp.int32), bid])

            m0 = col(0)
            m1 = col(1)
            m2 = col(2)
            m3 = col(3)
            gx = ((m0 + m2) * jnp.float32(0.5) - pcx) * rw * jnp.float32(10.0)
            gy = ((m1 + m3) * jnp.float32(0.5) - pcy) * rh * jnp.float32(10.0)
            gw = _logf((m2 - m0) * rw) * jnp.float32(5.0)
            gh = _logf((m3 - m1) * rh) * jnp.float32(5.0)
            lbox = (_smooth_l1(loc_b[0, sl], gx) +
                    _smooth_l1(loc_b[1, sl], gy) +
                    _smooth_l1(loc_b[2, sl], gw) +
                    _smooth_l1(loc_b[3, sl], gh))
            ab = ab + posf * lbox

            llm = jnp.zeros((16,), jnp.float32)
            for i in range(5):
                lx = col(4 + 2 * i)
                ly = col(5 + 2 * i)
                ex = (lx - pcx) * rw * jnp.float32(10.0)
                ey = (ly - pcy) * rh * jnp.float32(10.0)
                llm = (llm + _smooth_l1(lm_b[2 * i, sl], ex) +
                       _smooth_l1(lm_b[2 * i + 1, sl], ey))
            al = al + posf * llm

            x0 = conf_b[0, sl]
            x1 = conf_b[1, sl]
            z = jnp.where(pos, x0 - x1, x1 - x0)
            cev = (jnp.maximum(z, jnp.float32(0.0)) +
                   _logf(jnp.float32(1.0) + jnp.exp(-jnp.abs(z))))
            ac = ac + posf * cev
            an = an + posf
            valid = ((lo + p0 + base) + lax.iota(jnp.int32, 16)) < P_REAL
            ce[gsl] = jnp.where(pos | jnp.logical_not(valid),
                                jnp.float32(0.0), cev)
            return ab, al, ac, an

        return lax.fori_loop(0, NV, vloop, carry)

    zero = jnp.zeros((16,), jnp.float32)
    acc_box, acc_lm, acc_ce, acc_np = lax.fori_loop(
        0, NCHUNK, pass_b, (zero, zero, zero, zero))

    npos_f = _red16(acc_np, jnp.add)[0]
    loss_box = _red16(acc_box, jnp.add)[0]
    loss_lm = _red16(acc_lm, jnp.add)[0]
    ce_pos = _red16(acc_ce, jnp.add)[0]

    stgb[...] = jnp.full((16,), 1.0, jnp.float32) * npos_f
    plsc.subcore_barrier()
    pltpu.sync_copy(stgb, shb.at[sid])
    plsc.subcore_barrier()
    plsc.subcore_barrier()
    pltpu.sync_copy(shb.at[sid ^ 8], prtb)
    plsc.subcore_barrier()
    npos_tot = npos_f + prtb[pl.ds(0, 16)][0]

    k = jnp.minimum(npos_tot.astype(jnp.int32) * NEGPOS, P_REAL - 1)

    def bis(i, carry):
        lo_b, hi_b = carry
        mid = lo_b + ((hi_b - lo_b + 1) >> 1)
        midv = jnp.full((16,), 1, jnp.int32) * mid

        def cnt_loop(v, a):
            bits = plsc.bitcast(ce[pl.ds(v * 16, 16)], jnp.int32)
            return a + plsc.all_reduce_population_count(bits >= midv)

        cnt = lax.fori_loop(0, NVH, cnt_loop,
                            jnp.zeros((16,), jnp.int32))
        stgb[...] = jnp.full((16,), 1.0, jnp.float32) * cnt[0].astype(jnp.float32)
        plsc.subcore_barrier()
        pltpu.sync_copy(stgb, shb.at[sid])
        plsc.subcore_barrier()
        plsc.subcore_barrier()
        pltpu.sync_copy(shb.at[sid ^ 8], prtb)
        plsc.subcore_barrier()
        cnt_tot = cnt[0] + prtb[pl.ds(0, 16)][0].astype(jnp.int32)
        ge = cnt_tot >= k
        lo_b = jnp.where(ge, mid, lo_b)
        hi_b = jnp.where(ge, hi_b, mid - 1)
        return lo_b, hi_b

    lo_b, _ = lax.fori_loop(0, 31, bis,
                            (jnp.int32(0), jnp.int32(0x7F800000)))
    vkv = plsc.bitcast(jnp.full((16,), 1, jnp.int32) * lo_b, jnp.float32)

    def fin_loop(v, carry):
        sgt, cgt = carry
        x = ce[pl.ds(v * 16, 16)]
        gt = x > vkv
        sgt = sgt + jnp.where(gt, x, jnp.float32(0.0))
        cgt = cgt + plsc.all_reduce_population_count(gt)
        return sgt, cgt

    sgt, cgt = lax.fori_loop(
        0, NVH, fin_loop,
        (jnp.zeros((16,), jnp.float32), jnp.zeros((16,), jnp.int32)))
    sgt_sum = _red16(sgt, jnp.add)[0]

    li = lax.iota(jnp.int32, 16)
    fz = jnp.float32(0.0)
    o = (jnp.where(li == 0, npos_f, fz) +
         jnp.where(li == 1, loss_box, fz) +
         jnp.where(li == 2, ce_pos, fz) +
         jnp.where(li == 3, loss_lm, fz) +
         jnp.where(li == 4, sgt_sum, fz) +
         jnp.where(li == 5, cgt[0].astype(jnp.float32), fz) +
         jnp.where(li == 6, vkv, fz) +
         jnp.where(li == 7, k.astype(jnp.float32), fz))
    outv[pl.ds(0, 16)] = o
    pltpu.sync_copy(outv, out_h.at[cid * 16 + sid])


def kernel(conf_data, loc_data, landm_data, priors, targets):
    pad = P - P_REAL
    conf_t = jnp.pad(jnp.transpose(conf_data, (0, 2, 1)),
                     ((0, 0), (0, 0), (0, pad)))
    loc_t = jnp.pad(jnp.transpose(loc_data, (0, 2, 1)),
                    ((0, 0), (0, 0), (0, pad)))
    lm_t = jnp.pad(jnp.transpose(landm_data, (0, 2, 1)),
                   ((0, 0), (0, 0), (0, pad)))
    pri_pad = jnp.broadcast_to(
        jnp.array([[100.0], [100.0], [1.0], [1.0]], jnp.float32), (4, pad))
    pri_t = jnp.concatenate([jnp.transpose(priors, (1, 0)), pri_pad], axis=1)
    tgt_t = jnp.pad(jnp.transpose(targets, (0, 2, 1)),
                    ((0, 0), (0, 0), (0, 128 - G)))
    part = _sc_forward(conf_t, loc_t, lm_t, pri_t, tgt_t)
    v = part.reshape(2, 2, 8, 128)
    h0 = v[:, 0]
    h1 = v[:, 1]
    npos = jnp.maximum(jnp.sum(h0[..., 0] + h1[..., 0]), 1.0)
    loss_box = jnp.sum(h0[..., 1] + h1[..., 1])
    ce_pos = jnp.sum(h0[..., 2] + h1[..., 2])
    loss_lm = jnp.sum(h0[..., 3] + h1[..., 3])
    topk = jnp.sum(h0[..., 4] + h1[..., 4] +
                   (h0[..., 7] - h0[..., 5] - h1[..., 5]) * h0[..., 6])
    return (loss_box / npos, (ce_pos + topk) / npos, loss_lm / npos)


# speedup vs baseline: 1.0505x; 1.0348x over previous
import functools

import jax
import jax.numpy as jnp
from jax import lax
from jax.experimental import pallas as pl
from jax.experimental.pallas import tpu as pltpu
from jax.experimental.pallas import tpu_sc as plsc

P_REAL = 16800
P = 16896
HP = P // 2
G = 32
B = 16
CH = 1408
NCHUNK = HP // CH
NV = CH // 16
NVH = HP // 16
OVERLAP_T = 0.35
NEGPOS = 7
LN2 = 0.6931471805599453


def _logf(x):
    xb = plsc.bitcast(x, jnp.int32)
    e = ((xb >> 23) & 0xFF) - 127
    m = plsc.bitcast((xb & 0x007FFFFF) | (127 << 23), jnp.float32)
    big = m > jnp.float32(1.4142135)
    m = jnp.where(big, m * jnp.float32(0.5), m)
    e = jnp.where(big, e + 1, e)
    z = (m - jnp.float32(1.0)) / (m + jnp.float32(1.0))
    z2 = z * z
    p = jnp.float32(2.0 / 9.0)
    p = p * z2 + jnp.float32(2.0 / 7.0)
    p = p * z2 + jnp.float32(2.0 / 5.0)
    p = p * z2 + jnp.float32(2.0 / 3.0)
    p = p * z2 + jnp.float32(2.0)
    return e.astype(jnp.float32) * jnp.float32(LN2) + z * p


def _lane_rotate(x, s):
    idx = (lax.iota(jnp.int32, 16) + s) & 15
    return jnp.take(x, idx)


def _red16(x, op):
    for s in (1, 2, 4, 8):
        x = op(x, _lane_rotate(x, s))
    return x


def _smooth_l1(pred, tgt):
    d = jnp.abs(pred - tgt)
    return jnp.where(d < jnp.float32(1.0),
                     jnp.float32(0.5) * d * d,
                     d - jnp.float32(0.5))


_mesh = plsc.VectorSubcoreMesh(core_axis_name="c", subcore_axis_name="s")


@functools.partial(
    pl.kernel,
    out_type=jax.ShapeDtypeStruct((32, 128), jnp.float32),
    mesh=_mesh,
    compiler_params=pltpu.CompilerParams(needs_layout_passes=False),
    scratch_types=[
        pltpu.VMEM((4, HP), jnp.float32),
        pltpu.VMEM((2, CH), jnp.float32),
        pltpu.VMEM((4, CH), jnp.float32),
        pltpu.VMEM((10, CH), jnp.float32),
        pltpu.VMEM((15, 128), jnp.float32),
        pltpu.VMEM((HP,), jnp.float32),
        pltpu.VMEM((HP,), jnp.int32),
        pltpu.VMEM((HP,), jnp.float32),
        pltpu.VMEM((G, 16), jnp.float32),
        pltpu.VMEM((G, 16), jnp.int32),
        pltpu.VMEM((48,), jnp.float32),
        pltpu.VMEM((128,), jnp.float32),
        pltpu.VMEM((64,), jnp.float32),
        pltpu.VMEM((64,), jnp.float32),
        pltpu.VMEM((16,), jnp.float32),
        pltpu.VMEM((16,), jnp.float32),
        pltpu.VMEM_SHARED((16, 64), jnp.float32),
        pltpu.VMEM_SHARED((16, 16), jnp.float32),
    ],
)
def _sc_forward(conf_h, loc_h, lm_h, pri_h, tgt_h, out_h,
                pri_b, conf_b, loc_b, lm_b, tgt_b,
                bov, bidx, ce, gmax, gidx, tarea, outv,
                stga, prta, stgb, prtb,
                sha, shb):
    cid = lax.axis_index("c")
    sid = lax.axis_index("s")
    b = cid * 8 + (sid & 7)
    half = sid >> 3
    lo = half * HP

    pltpu.sync_copy(tgt_h.at[b], tgt_b)
    pltpu.sync_copy(pri_h.at[:, pl.ds(lo, HP)], pri_b)

    for h in range(2):
        sl = pl.ds(h * 16, 16)
        t0 = tgt_b[0, sl]
        t1 = tgt_b[1, sl]
        t2 = tgt_b[2, sl]
        t3 = tgt_b[3, sl]
        tarea[sl] = (t2 - t0) * (t3 - t1)

    def init_g(g, _):
        gmax[g, :] = jnp.full((16,), -1.0, jnp.float32)
        gidx[g, :] = jnp.zeros((16,), jnp.int32)
        return 0
    lax.fori_loop(0, G, init_g, 0)

    def vloop_a(v, _):
        base = v * 16
        pcx = pri_b[0, pl.ds(base, 16)]
        pcy = pri_b[1, pl.ds(base, 16)]
        pw = pri_b[2, pl.ds(base, 16)]
        ph = pri_b[3, pl.ds(base, 16)]
        px0 = pcx - pw * jnp.float32(0.5)
        py0 = pcy - ph * jnp.float32(0.5)
        px1 = pcx + pw * jnp.float32(0.5)
        py1 = pcy + ph * jnp.float32(0.5)
        parea = (px1 - px0) * (py1 - py0)
        pidx = (lo + base) + lax.iota(jnp.int32, 16)

        def gloop(g4, carry):
            bestv, bestg = carry
            for j in range(4):
                g = g4 * 4 + j
                gw16 = pl.ds(g, 16)
                tx0 = tgt_b[0, gw16][0]
                ty0 = tgt_b[1, gw16][0]
                tx1 = tgt_b[2, gw16][0]
                ty1 = tgt_b[3, gw16][0]
                ta = tarea[gw16][0]
                iw = jnp.maximum(
                    jnp.minimum(px1, tx1) - jnp.maximum(px0, tx0),
                    jnp.float32(0.0))
                ih = jnp.maximum(
                    jnp.minimum(py1, ty1) - jnp.maximum(py0, ty0),
                    jnp.float32(0.0))
                inter = iw * ih
                ov = inter / (ta + parea - inter)
                up = ov > bestv
                bestv = jnp.where(up, ov, bestv)
                bestg = jnp.where(up, g, bestg)
                gm = gmax[g, :]
                u2 = ov > gm
                gmax[g, :] = jnp.where(u2, ov, gm)
                gidx[g, :] = jnp.where(u2, pidx, gidx[g, :])
            return bestv, bestg

        bestv, bestg = lax.fori_loop(
            0, G // 4, gloop,
            (jnp.full((16,), -1.0, jnp.float32),
             jnp.zeros((16,), jnp.int32)))
        bov[pl.ds(base, 16)] = bestv
        bidx[pl.ds(base, 16)] = bestg
        return 0

    lax.fori_loop(0, NVH, vloop_a, 0)

    def bp_red(g, carry):
        mlo, mhi, ilo, ihi = carry
        gm = gmax[g, :]
        m = _red16(gm, jnp.maximum)
        cand = jnp.where(gm == m, gidx[g, :], jnp.int32(0x7FFFFFFF))
        bpg = _red16(cand, jnp.minimum)[0]
        l16 = lax.iota(jnp.int32, 16)
        mlo = jnp.where(l16 == g, m[0], mlo)
        mhi = jnp.where(l16 == (g - 16), m[0], mhi)
        ilo = jnp.where(l16 == g, bpg, ilo)
        ihi = jnp.where(l16 == (g - 16), bpg, ihi)
        return mlo, mhi, ilo, ihi

    mx_lo, mx_hi, bp_lo, bp_hi = lax.fori_loop(
        0, G, bp_red,
        (jnp.zeros((16,), jnp.float32), jnp.zeros((16,), jnp.float32),
         jnp.zeros((16,), jnp.int32), jnp.zeros((16,), jnp.int32)))

    stga[pl.ds(0, 16)] = mx_lo
    stga[pl.ds(16, 16)] = mx_hi
    stga[pl.ds(32, 16)] = bp_lo.astype(jnp.float32)
    stga[pl.ds(48, 16)] = bp_hi.astype(jnp.float32)
    plsc.subcore_barrier()
    pltpu.sync_copy(stga, sha.at[sid])
    plsc.subcore_barrier()
    plsc.subcore_barrier()
    pltpu.sync_copy(sha.at[sid ^ 8], prta)
    plsc.subcore_barrier()
    p_mx_lo = prta[pl.ds(0, 16)]
    p_mx_hi = prta[pl.ds(16, 16)]
    p_bp_lo = prta[pl.ds(32, 16)].astype(jnp.int32)
    p_bp_hi = prta[pl.ds(48, 16)].astype(jnp.int32)

    is_low = half == 0
    lmx_lo = jnp.where(is_low, mx_lo, p_mx_lo)
    hmx_lo = jnp.where(is_low, p_mx_lo, mx_lo)
    lbp_lo = jnp.where(is_low, bp_lo, p_bp_lo)
    hbp_lo = jnp.where(is_low, p_bp_lo, bp_lo)
    bp_lo = jnp.where(lmx_lo >= hmx_lo, lbp_lo, hbp_lo)
    lmx_hi = jnp.where(is_low, mx_hi, p_mx_hi)
    hmx_hi = jnp.where(is_low, p_mx_hi, mx_hi)
    lbp_hi = jnp.where(is_low, bp_hi, p_bp_hi)
    hbp_hi = jnp.where(is_low, p_bp_hi, bp_hi)
    bp_hi = jnp.where(lmx_hi >= hmx_hi, lbp_hi, hbp_hi)

    l16 = lax.iota(jnp.int32, 16)
    in_lo = (bp_lo >= lo) & (bp_lo < lo + HP)
    in_hi = (bp_hi >= lo) & (bp_hi < lo + HP)
    loc_lo = jnp.clip(bp_lo - lo, 0, HP - 1)
    loc_hi = jnp.clip(bp_hi - lo, 0, HP - 1)
    two = jnp.full((16,), 2.0, jnp.float32)
    plsc.store_scatter(bov, [loc_lo], two, mask=in_lo)
    plsc.store_scatter(bov, [loc_hi], two, mask=in_hi)
    for j in range(16):
        plsc.store_scatter(bidx, [loc_lo], l16, mask=in_lo & (l16 == j))
    for j in range(16):
        plsc.store_scatter(bidx, [loc_hi], l16 + 16, mask=in_hi & (l16 == j))

    def pass_b(ci, carry):
        acc_box, acc_lm, acc_ce, acc_np = carry
        p0 = ci * CH
        pltpu.sync_copy(conf_h.at[b, :, pl.ds(lo + p0, CH)], conf_b)
        pltpu.sync_copy(loc_h.at[b, :, pl.ds(lo + p0, CH)], loc_b)
        pltpu.sync_copy(lm_h.at[b, :, pl.ds(lo + p0, CH)], lm_b)

        def vloop(v, carry2):
            ab, al, ac, an = carry2
            base = v * 16
            sl = pl.ds(base, 16)
            gsl = pl.ds(p0 + base, 16)
            pcx = pri_b[0, gsl]
            pcy = pri_b[1, gsl]
            pw = pri_b[2, gsl]
            ph = pri_b[3, gsl]
            rw = jnp.float32(1.0) / pw
            rh = jnp.float32(1.0) / ph
            bid = bidx[gsl]
            pos = bov[gsl] >= jnp.float32(OVERLAP_T)
            posf = jnp.where(pos, jnp.float32(1.0), jnp.float32(0.0))

            def col(c):
                return plsc.load_gather(
                    tgt_b, [jnp.full((16,), c, jnp.int32), bid])

            m0 = col(0)
            m1 = col(1)
            m2 = col(2)
            m3 = col(3)
            gx = ((m0 + m2) * jnp.float32(0.5) - pcx) * rw * jnp.float32(10.0)
            gy = ((m1 + m3) * jnp.float32(0.5) - pcy) * rh * jnp.float32(10.0)
            gw = _logf((m2 - m0) * rw) * jnp.float32(5.0)
            gh = _logf((m3 - m1) * rh) * jnp.float32(5.0)
            lbox = (_smooth_l1(loc_b[0, sl], gx) +
                    _smooth_l1(loc_b[1, sl], gy) +
                    _smooth_l1(loc_b[2, sl], gw) +
                    _smooth_l1(loc_b[3, sl], gh))
            ab = ab + posf * lbox

            llm = jnp.zeros((16,), jnp.float32)
            for i in range(5):
                lx = col(4 + 2 * i)
                ly = col(5 + 2 * i)
                ex = (lx - pcx) * rw * jnp.float32(10.0)
                ey = (ly - pcy) * rh * jnp.float32(10.0)
                llm = (llm + _smooth_l1(lm_b[2 * i, sl], ex) +
                       _smooth_l1(lm_b[2 * i + 1, sl], ey))
            al = al + posf * llm

            x0 = conf_b[0, sl]
            x1 = conf_b[1, sl]
            z = jnp.where(pos, x0 - x1, x1 - x0)
            cev = (jnp.maximum(z, jnp.float32(0.0)) +
                   _logf(jnp.float32(1.0) + jnp.exp(-jnp.abs(z))))
            ac = ac + posf * cev
            an = an + posf
            valid = ((lo + p0 + base) + lax.iota(jnp.int32, 16)) < P_REAL
            ce[gsl] = jnp.where(pos | jnp.logical_not(valid),
                                jnp.float32(0.0), cev)
            return ab, al, ac, an

        return lax.fori_loop(0, NV, vloop, carry)

    zero = jnp.zeros((16,), jnp.float32)
    acc_box, acc_lm, acc_ce, acc_np = lax.fori_loop(
        0, NCHUNK, pass_b, (zero, zero, zero, zero))

    npos_f = _red16(acc_np, jnp.add)[0]
    loss_box = _red16(acc_box, jnp.add)[0]
    loss_lm = _red16(acc_lm, jnp.add)[0]
    ce_pos = _red16(acc_ce, jnp.add)[0]

    stgb[...] = jnp.full((16,), 1.0, jnp.float32) * npos_f
    plsc.subcore_barrier()
    pltpu.sync_copy(stgb, shb.at[sid])
    plsc.subcore_barrier()
    plsc.subcore_barrier()
    pltpu.sync_copy(shb.at[sid ^ 8], prtb)
    plsc.subcore_barrier()
    npos_tot = npos_f + prtb[pl.ds(0, 16)][0]

    k = jnp.minimum(npos_tot.astype(jnp.int32) * NEGPOS, P_REAL - 1)

    def bis(i, carry):
        lo_b, hi_b = carry
        mid = lo_b + ((hi_b - lo_b + 1) >> 1)
        midv = jnp.full((16,), 1, jnp.int32) * mid

        def cnt_loop(v, a):
            bits = plsc.bitcast(ce[pl.ds(v * 16, 16)], jnp.int32)
            return a + plsc.all_reduce_population_count(bits >= midv)

        cnt = lax.fori_loop(0, NVH, cnt_loop,
                            jnp.zeros((16,), jnp.int32))
        stgb[...] = jnp.full((16,), 1.0, jnp.float32) * cnt[0].astype(jnp.float32)
        plsc.subcore_barrier()
        pltpu.sync_copy(stgb, shb.at[sid])
        plsc.subcore_barrier()
        plsc.subcore_barrier()
        pltpu.sync_copy(shb.at[sid ^ 8], prtb)
        plsc.subcore_barrier()
        cnt_tot = cnt[0] + prtb[pl.ds(0, 16)][0].astype(jnp.int32)
        ge = cnt_tot >= k
        lo_b = jnp.where(ge, mid, lo_b)
        hi_b = jnp.where(ge, hi_b, mid - 1)
        return lo_b, hi_b

    lo_b, _ = lax.fori_loop(0, 31, bis,
                            (jnp.int32(0), jnp.int32(0x7F800000)))
    vkv = plsc.bitcast(jnp.full((16,), 1, jnp.int32) * lo_b, jnp.float32)

    def fin_loop(v, carry):
        sgt, cgt = carry
        x = ce[pl.ds(v * 16, 16)]
        gt = x > vkv
        sgt = sgt + jnp.where(gt, x, jnp.float32(0.0))
        cgt = cgt + plsc.all_reduce_population_count(gt)
        return sgt, cgt

    sgt, cgt = lax.fori_loop(
        0, NVH, fin_loop,
        (jnp.zeros((16,), jnp.float32), jnp.zeros((16,), jnp.int32)))
    sgt_sum = _red16(sgt, jnp.add)[0]

    li = lax.iota(jnp.int32, 16)
    fz = jnp.float32(0.0)
    o = (jnp.where(li == 0, npos_f, fz) +
         jnp.where(li == 1, loss_box, fz) +
         jnp.where(li == 2, ce_pos, fz) +
         jnp.where(li == 3, loss_lm, fz) +
         jnp.where(li == 4, sgt_sum, fz) +
         jnp.where(li == 5, cgt[0].astype(jnp.float32), fz) +
         jnp.where(li == 6, vkv, fz) +
         jnp.where(li == 7, k.astype(jnp.float32), fz))
    outv[pl.ds(0, 16)] = o
    pltpu.sync_copy(outv, out_h.at[cid * 16 + sid])


def kernel(conf_data, loc_data, landm_data, priors, targets):
    pad = P - P_REAL
    conf_t = jnp.pad(jnp.transpose(conf_data, (0, 2, 1)),
                     ((0, 0), (0, 0), (0, pad)))
    loc_t = jnp.pad(jnp.transpose(loc_data, (0, 2, 1)),
                    ((0, 0), (0, 0), (0, pad)))
    lm_t = jnp.pad(jnp.transpose(landm_data, (0, 2, 1)),
                   ((0, 0), (0, 0), (0, pad)))
    pri_pad = jnp.broadcast_to(
        jnp.array([[100.0], [100.0], [1.0], [1.0]], jnp.float32), (4, pad))
    pri_t = jnp.concatenate([jnp.transpose(priors, (1, 0)), pri_pad], axis=1)
    tgt_t = jnp.pad(jnp.transpose(targets, (0, 2, 1)),
                    ((0, 0), (0, 0), (0, 128 - G)))
    part = _sc_forward(conf_t, loc_t, lm_t, pri_t, tgt_t)
    v = part.reshape(2, 2, 8, 128)
    h0 = v[:, 0]
    h1 = v[:, 1]
    npos = jnp.maximum(jnp.sum(h0[..., 0] + h1[..., 0]), 1.0)
    loss_box = jnp.sum(h0[..., 1] + h1[..., 1])
    ce_pos = jnp.sum(h0[..., 2] + h1[..., 2])
    loss_lm = jnp.sum(h0[..., 3] + h1[..., 3])
    topk = jnp.sum(h0[..., 4] + h1[..., 4] +
                   (h0[..., 7] - h0[..., 5] - h1[..., 5]) * h0[..., 6])
    return (loss_box / npos, (ce_pos + topk) / npos, loss_lm / npos)
